# baseline (speedup 1.0000x reference)
import functools

import jax
import jax.numpy as jnp
from jax import lax
from jax.experimental import pallas as pl
from jax.experimental.pallas import tpu as pltpu
from jax.experimental.pallas import tpu_sc as plsc

N_NODES = 10000
N_EDGES = 320000
D_FEAT = 128
D_EDGE = 16
LN_EPS = 1e-6

NC = 2
NS = 16
NPAD = 10240
EPAD = 327680
EW = EPAD // (NC * NS)
BLK = 64
NBUF = 2
ITERS = EW // (NBUF * BLK)
ESTORE = EPAD + NBUF * BLK
ROWS_PER_TILE = NPAD // NS
WB = ROWS_PER_TILE // BLK


def _sc_segment_sums():
    mesh = plsc.VectorSubcoreMesh(core_axis_name="c", subcore_axis_name="s")

    @functools.partial(
        pl.kernel,
        out_type=[
            jax.ShapeDtypeStruct((NC * NPAD, D_FEAT), jnp.float32),
            jax.ShapeDtypeStruct((NC * NPAD, D_EDGE), jnp.float32),
        ],
        mesh=mesh,
        compiler_params=pltpu.CompilerParams(use_tc_tiling_on_sc=False),
        scratch_types=(
            [pltpu.VMEM((BLK,), jnp.int32) for _ in range(NBUF)]
            + [pltpu.VMEM((BLK,), jnp.int32) for _ in range(NBUF)]
            + [pltpu.VMEM((BLK, D_FEAT), jnp.float32) for _ in range(NBUF)]
            + [pltpu.VMEM((BLK, D_EDGE), jnp.float32) for _ in range(NBUF)]
            + [pltpu.VMEM_SHARED((NPAD, D_FEAT), jnp.float32),
               pltpu.VMEM_SHARED((NPAD, D_EDGE), jnp.float32)]
            + [pltpu.SemaphoreType.DMA] * (6 * NBUF)
        ),
    )
    def seg(nodes_hbm, edges_hbm, send_hbm, recv_hbm, out_n, out_e, *scr):
        p = 0
        sidx = scr[p:p + NBUF]; p += NBUF
        ridx = scr[p:p + NBUF]; p += NBUF
        rows = scr[p:p + NBUF]; p += NBUF
        erow = scr[p:p + NBUF]; p += NBUF
        acc = scr[p]; eacc = scr[p + 1]; p += 2
        sisem = scr[p:p + NBUF]; p += NBUF
        risem = scr[p:p + NBUF]; p += NBUF
        gsem = scr[p:p + NBUF]; p += NBUF
        esem = scr[p:p + NBUF]; p += NBUF
        scsem = scr[p:p + NBUF]; p += NBUF
        escsem = scr[p:p + NBUF]; p += NBUF

        c = lax.axis_index("c")
        s = lax.axis_index("s")

        zero16 = jnp.zeros((16,), jnp.float32)

        def zrow(i, carry):
            for j in range(D_FEAT // 16):
                rows[0][i, pl.ds(j * 16, 16)] = zero16
            erow[0][i, :] = zero16
            return carry

        lax.fori_loop(0, BLK, zrow, 0)
        for t in range(WB):
            off = s * ROWS_PER_TILE + t * BLK
            pltpu.sync_copy(rows[0], acc.at[pl.ds(off, BLK)])
            pltpu.sync_copy(erow[0], eacc.at[pl.ds(off, BLK)])
        plsc.subcore_barrier()

        base_t = (s * NC + c) * EW

        def fire_idx(j, blk):
            off = base_t + blk * BLK
            pltpu.async_copy(send_hbm.at[pl.ds(off, BLK)], sidx[j], sisem[j])
            pltpu.async_copy(recv_hbm.at[pl.ds(off, BLK)], ridx[j], risem[j])
            data_off = jnp.minimum(off, N_EDGES - BLK)
            pltpu.async_copy(edges_hbm.at[pl.ds(data_off, BLK)],
                             erow[j], esem[j])

        for j in range(NBUF):
            fire_idx(j, j)

        def body(i, carry):
            for j in range(NBUF):
                pltpu.make_async_copy(send_hbm.at[pl.ds(0, BLK)],
                                      sidx[j], sisem[j]).wait()
                pltpu.async_copy(nodes_hbm.at[sidx[j]], rows[j], gsem[j])
            for j in range(NBUF):
                pltpu.make_async_copy(recv_hbm.at[pl.ds(0, BLK)],
                                      ridx[j], risem[j]).wait()
                pltpu.make_async_copy(edges_hbm.at[pl.ds(0, BLK)],
                                      erow[j], esem[j]).wait()
                pltpu.async_copy(erow[j], eacc.at[ridx[j]], escsem[j],
                                 add=True)
                pltpu.make_async_copy(nodes_hbm.at[sidx[j]], rows[j],
                                      gsem[j]).wait()
                pltpu.async_copy(rows[j], acc.at[ridx[j]], scsem[j], add=True)
            for j in range(NBUF):
                pltpu.make_async_copy(rows[j], acc.at[ridx[j]],
                                      scsem[j]).wait()
                pltpu.make_async_copy(erow[j], eacc.at[ridx[j]],
                                      escsem[j]).wait()
                fire_idx(j, (i + 1) * NBUF + j)
            return carry

        lax.fori_loop(0, ITERS, body, 0)
        for j in range(NBUF):
            pltpu.make_async_copy(send_hbm.at[pl.ds(0, BLK)],
                                  sidx[j], sisem[j]).wait()
            pltpu.make_async_copy(recv_hbm.at[pl.ds(0, BLK)],
                                  ridx[j], risem[j]).wait()
            pltpu.make_async_copy(edges_hbm.at[pl.ds(0, BLK)],
                                  erow[j], esem[j]).wait()
        plsc.subcore_barrier()

        for t in range(WB):
            off = s * ROWS_PER_TILE + t * BLK
            pltpu.sync_copy(acc.at[pl.ds(off, BLK)], rows[0])
            pltpu.sync_copy(rows[0], out_n.at[pl.ds(c * NPAD + off, BLK)])
            pltpu.sync_copy(eacc.at[pl.ds(off, BLK)], erow[0])
            pltpu.sync_copy(erow[0], out_e.at[pl.ds(c * NPAD + off, BLK)])

    return seg


def _tc_body(nodes_ref, sn_ref, se_ref, wm_ref, wnode_ref, w1_ref, b1_ref,
             w2_ref, b2_ref, lns_ref, lnb_ref, out_ref):
    f32 = jnp.float32
    hi = jax.lax.Precision.HIGHEST
    n = nodes_ref[...]
    sn = sn_ref[0] + sn_ref[1]
    se = se_ref[0] + se_ref[1]
    agg = (jnp.dot(sn, wm_ref[:D_FEAT, :], precision=hi, preferred_element_type=f32)
           + jnp.dot(se, wm_ref[D_FEAT:, :], precision=hi, preferred_element_type=f32))
    h = (jnp.dot(n, w1_ref[:D_FEAT, :], precision=hi, preferred_element_type=f32)
         + jnp.dot(agg, w1_ref[D_FEAT:, :], precision=hi, preferred_element_type=f32)
         + b1_ref[...])
    h = jnp.maximum(h, 0.0)
    pre = (jnp.dot(h, w2_ref[...], precision=hi, preferred_element_type=f32)
           + b2_ref[...]
           + jnp.dot(n, wnode_ref[...], precision=hi, preferred_element_type=f32))
    mean = jnp.mean(pre, axis=-1, keepdims=True)
    cen = pre - mean
    var = jnp.mean(cen * cen, axis=-1, keepdims=True)
    out_ref[...] = cen * jax.lax.rsqrt(var + LN_EPS) * lns_ref[...] + lnb_ref[...]


def kernel(nodes, edges, W_message, W_node, mlp_W1, mlp_b1, mlp_W2, mlp_b2,
           ln_scale, ln_bias, senders, receivers):
    f32 = jnp.float32
    npad_e = ESTORE - N_EDGES
    senders_p = jnp.concatenate([senders, jnp.zeros((npad_e,), jnp.int32)])
    receivers_p = jnp.concatenate(
        [receivers, jnp.full((npad_e,), N_NODES, jnp.int32)])

    sums_n, sums_e = _sc_segment_sums()(nodes, edges, senders_p, receivers_p)
    sums_n = sums_n.reshape(NC, NPAD, D_FEAT)
    sums_e = sums_e.reshape(NC, NPAD, D_EDGE)

    nodes_p = jnp.pad(nodes, ((0, NPAD - N_NODES), (0, 0)))

    bm = 512
    grid = NPAD // bm
    wcol = lambda i: (0, 0)
    out = pl.pallas_call(
        _tc_body,
        grid=(grid,),
        in_specs=[
            pl.BlockSpec((bm, D_FEAT), lambda i: (i, 0)),
            pl.BlockSpec((NC, bm, D_FEAT), lambda i: (0, i, 0)),
            pl.BlockSpec((NC, bm, D_EDGE), lambda i: (0, i, 0)),
            pl.BlockSpec((D_FEAT + D_EDGE, D_FEAT), wcol),
            pl.BlockSpec((D_FEAT, D_FEAT), wcol),
            pl.BlockSpec((2 * D_FEAT, D_FEAT), wcol),
            pl.BlockSpec((1, D_FEAT), wcol),
            pl.BlockSpec((D_FEAT, D_FEAT), wcol),
            pl.BlockSpec((1, D_FEAT), wcol),
            pl.BlockSpec((1, D_FEAT), wcol),
            pl.BlockSpec((1, D_FEAT), wcol),
        ],
        out_specs=pl.BlockSpec((bm, D_FEAT), lambda i: (i, 0)),
        out_shape=jax.ShapeDtypeStruct((NPAD, D_FEAT), f32),
    )(nodes_p, sums_n, sums_e, W_message, W_node, mlp_W1,
      mlp_b1.reshape(1, -1), mlp_W2, mlp_b2.reshape(1, -1),
      ln_scale.reshape(1, -1), ln_bias.reshape(1, -1))
    return out[:N_NODES]


# speedup vs baseline: 1.2923x; 1.2923x over previous
import functools

import jax
import jax.numpy as jnp
from jax import lax
from jax.experimental import pallas as pl
from jax.experimental.pallas import tpu as pltpu
from jax.experimental.pallas import tpu_sc as plsc

N_NODES = 10000
N_EDGES = 320000
D_FEAT = 128
D_HALF = D_FEAT // 2
D_EDGE = 16
LN_EPS = 1e-6

NC = 2
NS = 16
NPAD = 10240
EPAD = 327680
ESTORE = EPAD + 512
EW = EPAD // NS
BLK = 128
NBUF = 4
EBUF = 2
ITERS = EW // (NBUF * BLK)
ECHUNK = EW // 2
ROWS_PER_TILE = NPAD // NS


def _sc_segment_sums():
    mesh = plsc.VectorSubcoreMesh(core_axis_name="c", subcore_axis_name="s")

    @functools.partial(
        pl.kernel,
        out_type=[
            jax.ShapeDtypeStruct((NC * NPAD, D_HALF), jnp.float32),
            jax.ShapeDtypeStruct((NC * NPAD, D_EDGE), jnp.float32),
        ],
        mesh=mesh,
        compiler_params=pltpu.CompilerParams(use_tc_tiling_on_sc=False),
        scratch_types=(
            [pltpu.VMEM((BLK,), jnp.int32) for _ in range(NBUF)]
            + [pltpu.VMEM((BLK,), jnp.int32) for _ in range(NBUF)]
            + [pltpu.VMEM((BLK, D_HALF), jnp.float32) for _ in range(NBUF)]
            + [pltpu.VMEM((BLK,), jnp.int32) for _ in range(EBUF)]
            + [pltpu.VMEM((BLK, D_EDGE), jnp.float32) for _ in range(EBUF)]
            + [pltpu.VMEM_SHARED((NPAD, D_HALF), jnp.float32),
               pltpu.VMEM_SHARED((NPAD, D_EDGE), jnp.float32)]
            + [pltpu.SemaphoreType.DMA] * (4 * NBUF + 3 * EBUF)
        ),
    )
    def seg(nodes_hbm, edges_hbm, send2_hbm, recv_hbm, out_n, out_e, *scr):
        p = 0
        sidx = scr[p:p + NBUF]; p += NBUF
        ridx = scr[p:p + NBUF]; p += NBUF
        rows = scr[p:p + NBUF]; p += NBUF
        eidx = scr[p:p + EBUF]; p += EBUF
        erow = scr[p:p + EBUF]; p += EBUF
        acc = scr[p]; eacc = scr[p + 1]; p += 2
        sisem = scr[p:p + NBUF]; p += NBUF
        risem = scr[p:p + NBUF]; p += NBUF
        gsem = scr[p:p + NBUF]; p += NBUF
        scsem = scr[p:p + NBUF]; p += NBUF
        eisem = scr[p:p + EBUF]; p += EBUF
        edsem = scr[p:p + EBUF]; p += EBUF
        escsem = scr[p:p + EBUF]; p += EBUF

        c = lax.axis_index("c")
        s = lax.axis_index("s")

        zero16 = jnp.zeros((16,), jnp.float32)

        def zrow(i, carry):
            for j in range(D_HALF // 16):
                rows[0][i, pl.ds(j * 16, 16)] = zero16
            erow[0][i, :] = zero16
            return carry

        lax.fori_loop(0, BLK, zrow, 0)
        for t in range(ROWS_PER_TILE // BLK):
            off = s * ROWS_PER_TILE + t * BLK
            pltpu.sync_copy(rows[0], acc.at[pl.ds(off, BLK)])
            pltpu.sync_copy(erow[0], eacc.at[pl.ds(off, BLK)])
        plsc.subcore_barrier()

        nbase = s * EW
        sbase = c * ESTORE + nbase
        ebase = (2 * s + c) * ECHUNK

        def fire_nidx(j, blk):
            off = blk * BLK
            a = pltpu.async_copy(send2_hbm.at[pl.ds(sbase + off, BLK)],
                                 sidx[j], sisem[j])
            b = pltpu.async_copy(recv_hbm.at[pl.ds(nbase + off, BLK)],
                                 ridx[j], risem[j])
            return a, b

        def fire_eidx(j, blk):
            off = blk * BLK
            data_off = jnp.minimum(ebase + off, N_EDGES - BLK)
            a = pltpu.async_copy(recv_hbm.at[pl.ds(ebase + off, BLK)],
                                 eidx[j], eisem[j])
            b = pltpu.async_copy(edges_hbm.at[pl.ds(data_off, BLK)],
                                 erow[j], edsem[j])
            return a, b

        for j in range(NBUF):
            fire_nidx(j, j)
        for j in range(EBUF):
            fire_eidx(j, j)

        def body(i, carry):
            for j in range(NBUF):
                pltpu.make_async_copy(send2_hbm.at[pl.ds(0, BLK)],
                                      sidx[j], sisem[j]).wait()
                pltpu.async_copy(nodes_hbm.at[sidx[j]], rows[j], gsem[j])
            for j in range(EBUF):
                pltpu.make_async_copy(recv_hbm.at[pl.ds(0, BLK)],
                                      eidx[j], eisem[j]).wait()
                pltpu.make_async_copy(edges_hbm.at[pl.ds(0, BLK)],
                                      erow[j], edsem[j]).wait()
                pltpu.async_copy(erow[j], eacc.at[eidx[j]], escsem[j],
                                 add=True)
            for j in range(NBUF):
                pltpu.make_async_copy(nodes_hbm.at[sidx[j]], rows[j],
                                      gsem[j]).wait()
                pltpu.make_async_copy(recv_hbm.at[pl.ds(0, BLK)],
                                      ridx[j], risem[j]).wait()
                pltpu.async_copy(rows[j], acc.at[ridx[j]], scsem[j], add=True)
            for j in range(NBUF):
                pltpu.make_async_copy(rows[j], acc.at[ridx[j]],
                                      scsem[j]).wait()
                fire_nidx(j, (i + 1) * NBUF + j)
            for j in range(EBUF):
                pltpu.make_async_copy(erow[j], eacc.at[eidx[j]],
                                      escsem[j]).wait()
                fire_eidx(j, (i + 1) * EBUF + j)
            return carry

        lax.fori_loop(0, ITERS, body, 0)
        for j in range(NBUF):
            pltpu.make_async_copy(send2_hbm.at[pl.ds(0, BLK)],
                                  sidx[j], sisem[j]).wait()
            pltpu.make_async_copy(recv_hbm.at[pl.ds(0, BLK)],
                                  ridx[j], risem[j]).wait()
        for j in range(EBUF):
            pltpu.make_async_copy(recv_hbm.at[pl.ds(0, BLK)],
                                  eidx[j], eisem[j]).wait()
            pltpu.make_async_copy(edges_hbm.at[pl.ds(0, BLK)],
                                  erow[j], edsem[j]).wait()
        plsc.subcore_barrier()

        for t in range(ROWS_PER_TILE // BLK):
            off = s * ROWS_PER_TILE + t * BLK
            pltpu.sync_copy(acc.at[pl.ds(off, BLK)], rows[0])
            pltpu.sync_copy(rows[0], out_n.at[pl.ds(c * NPAD + off, BLK)])
            pltpu.sync_copy(eacc.at[pl.ds(off, BLK)], erow[0])
            pltpu.sync_copy(erow[0], out_e.at[pl.ds(c * NPAD + off, BLK)])

    return seg


def _tc_body(nodes_ref, sn_ref, se_ref, wm_ref, wnode_ref, w1_ref, b1_ref,
             w2_ref, b2_ref, lns_ref, lnb_ref, out_ref):
    f32 = jnp.float32
    hi = jax.lax.Precision.HIGHEST
    n = nodes_ref[...]
    sn = jnp.concatenate([sn_ref[0], sn_ref[1]], axis=-1)
    se = se_ref[0] + se_ref[1]
    agg = (jnp.dot(sn, wm_ref[:D_FEAT, :], precision=hi, preferred_element_type=f32)
           + jnp.dot(se, wm_ref[D_FEAT:, :], precision=hi, preferred_element_type=f32))
    h = (jnp.dot(n, w1_ref[:D_FEAT, :], precision=hi, preferred_element_type=f32)
         + jnp.dot(agg, w1_ref[D_FEAT:, :], precision=hi, preferred_element_type=f32)
         + b1_ref[...])
    h = jnp.maximum(h, 0.0)
    pre = (jnp.dot(h, w2_ref[...], precision=hi, preferred_element_type=f32)
           + b2_ref[...]
           + jnp.dot(n, wnode_ref[...], precision=hi, preferred_element_type=f32))
    mean = jnp.mean(pre, axis=-1, keepdims=True)
    cen = pre - mean
    var = jnp.mean(cen * cen, axis=-1, keepdims=True)
    out_ref[...] = cen * jax.lax.rsqrt(var + LN_EPS) * lns_ref[...] + lnb_ref[...]


def kernel(nodes, edges, W_message, W_node, mlp_W1, mlp_b1, mlp_W2, mlp_b2,
           ln_scale, ln_bias, senders, receivers):
    f32 = jnp.float32
    npad_e = ESTORE - N_EDGES
    senders_p = jnp.concatenate([senders, jnp.zeros((npad_e,), jnp.int32)])
    receivers_p = jnp.concatenate(
        [receivers, jnp.full((npad_e,), N_NODES, jnp.int32)])
    senders2 = jnp.concatenate([senders_p, senders_p + N_NODES])
    nodes_split = jnp.concatenate([nodes[:, :D_HALF], nodes[:, D_HALF:]], axis=0)

    sums_n, sums_e = _sc_segment_sums()(
        nodes_split, edges, senders2, receivers_p)
    sums_n = sums_n.reshape(NC, NPAD, D_HALF)
    sums_e = sums_e.reshape(NC, NPAD, D_EDGE)

    nodes_p = jnp.pad(nodes, ((0, NPAD - N_NODES), (0, 0)))

    bm = 512
    grid = NPAD // bm
    wcol = lambda i: (0, 0)
    out = pl.pallas_call(
        _tc_body,
        grid=(grid,),
        in_specs=[
            pl.BlockSpec((bm, D_FEAT), lambda i: (i, 0)),
            pl.BlockSpec((NC, bm, D_HALF), lambda i: (0, i, 0)),
            pl.BlockSpec((NC, bm, D_EDGE), lambda i: (0, i, 0)),
            pl.BlockSpec((D_FEAT + D_EDGE, D_FEAT), wcol),
            pl.BlockSpec((D_FEAT, D_FEAT), wcol),
            pl.BlockSpec((2 * D_FEAT, D_FEAT), wcol),
            pl.BlockSpec((1, D_FEAT), wcol),
            pl.BlockSpec((D_FEAT, D_FEAT), wcol),
            pl.BlockSpec((1, D_FEAT), wcol),
            pl.BlockSpec((1, D_FEAT), wcol),
            pl.BlockSpec((1, D_FEAT), wcol),
        ],
        out_specs=pl.BlockSpec((bm, D_FEAT), lambda i: (i, 0)),
        out_shape=jax.ShapeDtypeStruct((NPAD, D_FEAT), f32),
    )(nodes_p, sums_n, sums_e, W_message, W_node, mlp_W1,
      mlp_b1.reshape(1, -1), mlp_W2, mlp_b2.reshape(1, -1),
      ln_scale.reshape(1, -1), ln_bias.reshape(1, -1))
    return out[:N_NODES]
